# Initial kernel scaffold; baseline (speedup 1.0000x reference)
import jax
import jax.numpy as jnp
from jax import lax
from jax.experimental import pallas as pl
from jax.experimental.pallas import tpu as pltpu

_BIG = jnp.int32(1 << 30)


def _scatter_kernel(radar_ref, mde_ref, out_ref):
    H, C = radar_ref.shape
    mde = mde_ref[...]
    mde_valid = mde != 0.0
    has_mde = jnp.any(mde_valid, axis=0, keepdims=True)
    posi = lax.broadcasted_iota(jnp.int32, (H, C), 0)

    def step(y, occ):
        d_r = radar_ref[pl.ds(y, 1), :]
        diffs = jnp.where(mde_valid, jnp.abs(mde - d_r), jnp.inf)
        m = jnp.min(diffs, axis=0, keepdims=True)
        bidx = jnp.min(jnp.where(diffs == m, posi, H), axis=0, keepdims=True)
        best = jnp.where(has_mde, bidx, y)
        key = 2 * jnp.abs(posi - best) - (posi > best).astype(jnp.int32)
        keyf = jnp.where(occ == 0.0, key, _BIG)
        km = jnp.min(keyf, axis=0, keepdims=True)
        fidx = jnp.min(jnp.where(keyf == km, posi, H), axis=0, keepdims=True)
        final = jnp.where(km < _BIG, fidx, best)
        write = d_r != 0.0
        return jnp.where((posi == final) & write, d_r, occ)

    out_ref[...] = lax.fori_loop(0, H, step, jnp.zeros((H, C), jnp.float32))


def kernel(radar_patches, mde_out_patches):
    W, B, C, H, _ = radar_patches.shape
    radar_flat = radar_patches[:, :, 0, :, 0]
    mde_flat = mde_out_patches[:, :, 0, :, 0]
    radar_t = radar_flat.reshape(W * B, H).T
    mde_t = mde_flat.reshape(W * B, H).T
    occ = pl.pallas_call(
        _scatter_kernel,
        out_shape=jax.ShapeDtypeStruct((H, W * B), jnp.float32),
    )(radar_t, mde_t)
    cols = occ.T.reshape(W, B, H)
    radar_gt = jnp.zeros((B, C, H, W), dtype=jnp.float32)
    radar_gt = radar_gt.at[:, 0, :, :].set(jnp.transpose(cols, (1, 2, 0)))
    return radar_gt


# speedup vs baseline: 283.0493x; 283.0493x over previous
import jax
import jax.numpy as jnp
from jax import lax
from jax.experimental import pallas as pl
from jax.experimental.pallas import tpu as pltpu

_BIG = 1 << 30


def _scatter_kernel(radar_ref, mde_ref, out_ref):
    H, C = radar_ref.shape
    mde = mde_ref[...]
    mde_valid = mde != 0.0
    has_mde = jnp.any(mde_valid, axis=0, keepdims=True)
    posi = lax.broadcasted_iota(jnp.int32, (H, C), 0)

    def step(y, occ):
        d_r = radar_ref[pl.ds(y, 1), :]
        diffs = jnp.where(mde_valid, jnp.abs(mde - d_r), jnp.inf)
        m = jnp.min(diffs, axis=0, keepdims=True)
        bidx = jnp.min(jnp.where(diffs == m, posi, H), axis=0, keepdims=True)
        best = jnp.where(has_mde, bidx, y)
        key = 2 * jnp.abs(posi - best) - (posi > best).astype(jnp.int32)
        keyf = jnp.where(occ == 0.0, key, _BIG)
        km = jnp.min(keyf, axis=0, keepdims=True)
        fidx = jnp.min(jnp.where(keyf == km, posi, H), axis=0, keepdims=True)
        final = jnp.where(km < _BIG, fidx, best)
        write = d_r != 0.0
        return jnp.where((posi == final) & write, d_r, occ)

    out_ref[...] = lax.fori_loop(0, H, step, jnp.zeros((H, C), jnp.float32))


def kernel(radar_patches, mde_out_patches):
    W, B, C, H, _ = radar_patches.shape
    radar_flat = radar_patches[:, :, 0, :, 0]
    mde_flat = mde_out_patches[:, :, 0, :, 0]
    radar_t = radar_flat.reshape(W * B, H).T
    mde_t = mde_flat.reshape(W * B, H).T
    occ = pl.pallas_call(
        _scatter_kernel,
        out_shape=jax.ShapeDtypeStruct((H, W * B), jnp.float32),
    )(radar_t, mde_t)
    cols = occ.T.reshape(W, B, H)
    radar_gt = jnp.zeros((B, C, H, W), dtype=jnp.float32)
    radar_gt = radar_gt.at[:, 0, :, :].set(jnp.transpose(cols, (1, 2, 0)))
    return radar_gt
